# Initial kernel scaffold; baseline (speedup 1.0000x reference)
import jax
import jax.numpy as jnp
from jax.experimental import pallas as pl


def kernel(positions, cell, pbc, species, sigma_matrix, epsilon_matrix, alpha_matrix, cutoff):
    raise NotImplementedError("write your pallas kernel here")


# speedup vs baseline: 2294.9422x; 2294.9422x over previous
import jax
import jax.numpy as jnp
from jax.experimental import pallas as pl
from jax.experimental.pallas import tpu as pltpu

_BI = 128
_PAD = 8


def _bilin(m):
    c0 = m[0, 0]
    c1 = m[1, 0] - m[0, 0]
    c2 = m[0, 1] - m[0, 0]
    c3 = m[1, 1] - m[1, 0] - m[0, 1] + m[0, 0]
    return c0, c1, c2, c3


def _pair_kernel(params_ref, row_ref, col_ref, out_ref):
    n = row_ref.shape[1]
    bi = col_ref.shape[0]
    pid = pl.program_id(0)

    cell = [[params_ref[3 * m + k] for k in range(3)] for m in range(3)]
    pbc = [params_ref[9 + m] for m in range(3)]
    cutoff = params_ref[12]
    sc = [params_ref[13 + t] for t in range(4)]
    ec = [params_ref[17 + t] for t in range(4)]
    ac = [params_ref[21 + t] for t in range(4)]

    dfrac = []
    for m in range(3):
        fi = col_ref[:, m].reshape(bi, 1)
        fj = row_ref[m, :].reshape(1, n)
        df = fj - fi
        df = df - jnp.round(df) * pbc[m]
        dfrac.append(df)

    dr = []
    for k in range(3):
        acc = dfrac[0] * cell[0][k]
        acc = acc + dfrac[1] * cell[1][k]
        acc = acc + dfrac[2] * cell[2][k]
        dr.append(acc)
    d2 = dr[0] * dr[0] + dr[1] * dr[1] + dr[2] * dr[2]

    i_glob = pid * bi + jax.lax.broadcasted_iota(jnp.int32, (bi, n), 0)
    j_glob = jax.lax.broadcasted_iota(jnp.int32, (bi, n), 1)
    eye = i_glob == j_glob

    d = jnp.sqrt(jnp.where(eye, 1.0, d2))

    si = col_ref[:, 3].reshape(bi, 1)
    sj = row_ref[3, :].reshape(1, n)
    sij = si * sj
    s = sc[0] + sc[1] * si + sc[2] * sj + sc[3] * sij
    e = ec[0] + ec[1] * si + ec[2] * sj + ec[3] * sij
    a = ac[0] + ac[1] * si + ac[2] * sj + ac[3] * sij

    inside = (d < cutoff) & (d < s) & jnp.logical_not(eye)
    b = jnp.where(inside, 1.0 - d / s, 0.5)
    lb = jnp.log(b)
    p = jnp.exp(a * lb)
    q = jnp.exp((a - 1.0) * lb)

    pe = jnp.where(inside, (e / a) * p, 0.0)
    coeff = jnp.where(inside, -(e / (s * d)) * q, 0.0)

    fx = jnp.sum(coeff * dr[0], axis=1).reshape(bi, 1)
    fy = jnp.sum(coeff * dr[1], axis=1).reshape(bi, 1)
    fz = jnp.sum(coeff * dr[2], axis=1).reshape(bi, 1)
    pes = jnp.sum(pe, axis=1).reshape(bi, 1)
    zeros = jnp.zeros((bi, 4), dtype=jnp.float32)
    out_ref[...] = jnp.concatenate([fx, fy, fz, pes, zeros], axis=1)


def kernel(positions, cell, pbc, species, sigma_matrix, epsilon_matrix, alpha_matrix, cutoff):
    n = positions.shape[0]
    inv_cell = jnp.linalg.inv(cell)
    frac = positions @ inv_cell
    spf = species.astype(jnp.float32)

    col = jnp.concatenate(
        [frac, spf[:, None], jnp.zeros((n, _PAD - 4), jnp.float32)], axis=1)
    row = col.T

    sc = _bilin(sigma_matrix)
    ec = _bilin(epsilon_matrix)
    ac = _bilin(alpha_matrix)
    params = jnp.zeros((32,), jnp.float32)
    params = params.at[0:9].set(cell.reshape(9).astype(jnp.float32))
    params = params.at[9:12].set(pbc.astype(jnp.float32))
    params = params.at[12].set(cutoff.astype(jnp.float32))
    params = params.at[13:17].set(jnp.stack(sc))
    params = params.at[17:21].set(jnp.stack(ec))
    params = params.at[21:25].set(jnp.stack(ac))

    grid = (n // _BI,)
    out = pl.pallas_call(
        _pair_kernel,
        grid=grid,
        in_specs=[
            pl.BlockSpec(memory_space=pltpu.SMEM),
            pl.BlockSpec((_PAD, n), lambda i: (0, 0)),
            pl.BlockSpec((_BI, _PAD), lambda i: (i, 0)),
        ],
        out_specs=pl.BlockSpec((_BI, _PAD), lambda i: (i, 0)),
        out_shape=jax.ShapeDtypeStruct((n, _PAD), jnp.float32),
    )(params, row, col)

    forces = out[:, :3]
    energy = 0.5 * jnp.sum(out[:, 3])
    return energy, forces
